# Initial kernel scaffold; baseline (speedup 1.0000x reference)
import jax
import jax.numpy as jnp
from jax.experimental import pallas as pl


def kernel(points_xyz, center_xyz, features):
    raise NotImplementedError("write your pallas kernel here")


# speedup vs baseline: 15.0357x; 15.0357x over previous
import jax
import jax.numpy as jnp
from jax import lax
from jax.experimental import pallas as pl
from jax.experimental.pallas import tpu as pltpu
from jax.experimental.pallas import tpu_sc as plsc

_B, _N, _M, _K, _C = 4, 8192, 1024, 32, 128
_R2 = 0.2 * 0.2
_NC, _NS, _L = 2, 16, 16
_NW = _NC * _NS
_TPB = _NW // _B
_MPT = _M // _TPB
_CPT = _C // _TPB
_RG = 8
_NG = _CPT // _RG
_CH = 2048
_MK = _M * _K
_NCH = 3 + _C

_mesh = plsc.VectorSubcoreMesh(core_axis_name="c", subcore_axis_name="s")


def _ball_query_body(ptsT, cenT, idx_out, xyz_out,
                     px, py, pz, cx, cy, cz, buf, idxb, gx, gy, gz):
    wid = lax.axis_index("s") * _NC + lax.axis_index("c")
    b = wid // _TPB
    ms = (wid % _TPB) * _MPT
    pltpu.sync_copy(ptsT.at[pl.ds((b * 3 + 0) * _N, _N)], px)
    pltpu.sync_copy(ptsT.at[pl.ds((b * 3 + 1) * _N, _N)], py)
    pltpu.sync_copy(ptsT.at[pl.ds((b * 3 + 2) * _N, _N)], pz)
    pltpu.sync_copy(cenT.at[pl.ds((b * 3 + 0) * _M + ms, _MPT)], cx)
    pltpu.sync_copy(cenT.at[pl.ds((b * 3 + 1) * _M + ms, _MPT)], cy)
    pltpu.sync_copy(cenT.at[pl.ds((b * 3 + 2) * _M + ms, _MPT)], cz)

    iota = lax.iota(jnp.int32, _L)

    def per_center(ml, carry):
        msplat = jnp.full((_L,), ml, jnp.int32)
        cxv = plsc.load_gather(cx, [msplat])
        cyv = plsc.load_gather(cy, [msplat])
        czv = plsc.load_gather(cz, [msplat])
        buf[pl.ds(0, _L)] = jnp.zeros((_L,), jnp.int32)

        def cond(st):
            pos, cnt = st
            return jnp.logical_and(cnt < _K, pos < _N)

        def body(st):
            pos, cnt = st
            dx = px[pl.ds(pos, _L)] - cxv
            dy = py[pl.ds(pos, _L)] - cyv
            dz = pz[pl.ds(pos, _L)] - czv
            d2 = dx * dx + dy * dy + dz * dz
            msk = d2 < _R2
            plsc.store_compressed(buf.at[pl.ds(cnt, _L)], iota + pos, mask=msk)
            cnt = cnt + jnp.sum(msk.astype(jnp.int32))
            return pos + _L, cnt

        _, cnt = lax.while_loop(cond, body, (jnp.int32(0), jnp.int32(0)))

        first = plsc.load_gather(buf, [jnp.zeros((_L,), jnp.int32)])
        for h in range(_K // _L):
            v = buf[pl.ds(h * _L, _L)]
            v = jnp.where((iota + h * _L) < cnt, v, first)
            o = ml * _K + h * _L
            idxb[pl.ds(o, _L)] = v
            gx[pl.ds(o, _L)] = plsc.load_gather(px, [v]) - cxv
            gy[pl.ds(o, _L)] = plsc.load_gather(py, [v]) - cyv
            gz[pl.ds(o, _L)] = plsc.load_gather(pz, [v]) - czv
        return carry

    lax.fori_loop(0, _MPT, per_center, 0)

    pltpu.sync_copy(idxb, idx_out.at[pl.ds(b * _MK + ms * _K, _MPT * _K)])
    pltpu.sync_copy(gx, xyz_out.at[pl.ds((b * 3 + 0) * _MK + ms * _K, _MPT * _K)])
    pltpu.sync_copy(gy, xyz_out.at[pl.ds((b * 3 + 1) * _MK + ms * _K, _MPT * _K)])
    pltpu.sync_copy(gz, xyz_out.at[pl.ds((b * 3 + 2) * _MK + ms * _K, _MPT * _K)])


def _group_body(feat, idx_all, xyz, out, idxv, frows, obuf, xbuf):
    wid = lax.axis_index("s") * _NC + lax.axis_index("c")
    b = wid // _TPB
    cg = wid % _TPB
    c0 = cg * _CPT

    pltpu.sync_copy(idx_all.at[pl.ds(b * _MK, _MK)], idxv)

    @pl.when(cg < 3)
    def _():
        def cpy(h, carry):
            src = (b * 3 + cg) * _MK + h * 8192
            dst = (b * _NCH + cg) * _MK + h * 8192
            pltpu.sync_copy(xyz.at[pl.ds(src, 8192)], xbuf)
            pltpu.sync_copy(xbuf, out.at[pl.ds(dst, 8192)])
            return carry
        lax.fori_loop(0, _MK // 8192, cpy, 0)

    for g in range(_NG):
        r0 = c0 + g * _RG
        pltpu.sync_copy(feat.at[pl.ds((b * _C + r0) * _N, _RG * _N)], frows)

        def chunk(jci, carry):
            jc = jci * _CH

            def jvec(i, carry2):
                iv = idxv[pl.ds(jc + i * _L, _L)]
                for r in range(_RG):
                    obuf[pl.ds(r * _CH + i * _L, _L)] = (
                        plsc.load_gather(frows, [iv + r * _N]))
                return carry2

            lax.fori_loop(0, _CH // _L, jvec, 0)
            for r in range(_RG):
                dst = (b * _NCH + 3 + r0 + r) * _MK + jc
                pltpu.sync_copy(obuf.at[pl.ds(r * _CH, _CH)],
                                out.at[pl.ds(dst, _CH)])
            return carry

        lax.fori_loop(0, _MK // _CH, chunk, 0)


_phase1 = pl.kernel(
    _ball_query_body,
    out_type=(
        jax.ShapeDtypeStruct((_B * _MK,), jnp.int32),
        jax.ShapeDtypeStruct((_B * 3 * _MK,), jnp.float32),
    ),
    mesh=_mesh,
    compiler_params=pltpu.CompilerParams(needs_layout_passes=False),
    scratch_types=[
        pltpu.VMEM((_N,), jnp.float32),
        pltpu.VMEM((_N,), jnp.float32),
        pltpu.VMEM((_N,), jnp.float32),
        pltpu.VMEM((_MPT,), jnp.float32),
        pltpu.VMEM((_MPT,), jnp.float32),
        pltpu.VMEM((_MPT,), jnp.float32),
        pltpu.VMEM((_K + _L,), jnp.int32),
        pltpu.VMEM((_MPT * _K,), jnp.int32),
        pltpu.VMEM((_MPT * _K,), jnp.float32),
        pltpu.VMEM((_MPT * _K,), jnp.float32),
        pltpu.VMEM((_MPT * _K,), jnp.float32),
    ],
)

_phase2 = pl.kernel(
    _group_body,
    out_type=jax.ShapeDtypeStruct((_B * _NCH * _MK,), jnp.float32),
    mesh=_mesh,
    compiler_params=pltpu.CompilerParams(needs_layout_passes=False),
    scratch_types=[
        pltpu.VMEM((_MK,), jnp.int32),
        pltpu.VMEM((_RG * _N,), jnp.float32),
        pltpu.VMEM((_RG * _CH,), jnp.float32),
        pltpu.VMEM((8192,), jnp.float32),
    ],
)


def kernel(points_xyz, center_xyz, features):
    ptsT = jnp.transpose(points_xyz, (0, 2, 1)).reshape(-1)
    cenT = jnp.transpose(center_xyz, (0, 2, 1)).reshape(-1)
    idx_all, xyz = _phase1(ptsT, cenT)
    out = _phase2(features.reshape(-1), idx_all, xyz)
    return out.reshape(_B, _NCH, _M, _K)


# speedup vs baseline: 24.2542x; 1.6131x over previous
import jax
import jax.numpy as jnp
from jax import lax
from jax.experimental import pallas as pl
from jax.experimental.pallas import tpu as pltpu
from jax.experimental.pallas import tpu_sc as plsc

_B, _N, _M, _K, _C = 4, 8192, 1024, 32, 128
_R2 = 0.2 * 0.2
_NC, _NS, _L = 2, 16, 16
_TPB = (_NC * _NS) // _B
_MPT = _M // _TPB
_MK = _M * _K
_NCH = 3 + _C

_PX, _PY, _PZ = 0, _N, 2 * _N
_CX = 3 * _N

_mesh = plsc.VectorSubcoreMesh(core_axis_name="c", subcore_axis_name="s")


def _body(ptsT, cenT, ff, idx_out, out4, frows, idxv, obuf, sem_out):
    wid = lax.axis_index("c") * _NS + lax.axis_index("s")
    b = wid // _TPB
    cg = wid % _TPB
    ms = cg * _MPT

    iota = lax.iota(jnp.int32, _L)
    zeros = jnp.zeros((_L,), jnp.int32)

    pltpu.sync_copy(ptsT.at[pl.ds((b * 3 + 0) * _N, _N)], frows.at[pl.ds(_PX, _N)])
    pltpu.sync_copy(ptsT.at[pl.ds((b * 3 + 1) * _N, _N)], frows.at[pl.ds(_PY, _N)])
    pltpu.sync_copy(ptsT.at[pl.ds((b * 3 + 2) * _N, _N)], frows.at[pl.ds(_PZ, _N)])
    for c3 in range(3):
        pltpu.sync_copy(cenT.at[pl.ds((b * 3 + c3) * _M + ms, _MPT)],
                        frows.at[pl.ds(_CX + c3 * _MPT, _MPT)])

    def per_center(ml, carry):
        cxv = plsc.load_gather(frows, [jnp.full((_L,), _CX + ml, jnp.int32)])
        cyv = plsc.load_gather(frows, [jnp.full((_L,), _CX + _MPT + ml, jnp.int32)])
        czv = plsc.load_gather(frows, [jnp.full((_L,), _CX + 2 * _MPT + ml, jnp.int32)])
        idxv[pl.ds(0, _L)] = zeros

        def cond(st):
            pos, cnt = st
            return jnp.logical_and(cnt < _K, pos < _N)

        def body(st):
            pos, cnt = st
            dxa = frows[pl.ds(_PX + pos, _L)] - cxv
            dya = frows[pl.ds(_PY + pos, _L)] - cyv
            dza = frows[pl.ds(_PZ + pos, _L)] - czv
            mska = dxa * dxa + dya * dya + dza * dza < _R2
            plsc.store_compressed(idxv.at[pl.ds(cnt, _L)], iota + pos, mask=mska)
            cnt1 = cnt + plsc.all_reduce_population_count(mska)[0]
            dxb = frows[pl.ds(_PX + pos + _L, _L)] - cxv
            dyb = frows[pl.ds(_PY + pos + _L, _L)] - cyv
            dzb = frows[pl.ds(_PZ + pos + _L, _L)] - czv
            mskb = dxb * dxb + dyb * dyb + dzb * dzb < _R2
            plsc.store_compressed(idxv.at[pl.ds(cnt1, _L)], iota + (pos + _L),
                                  mask=mskb)
            cnt2 = cnt1 + plsc.all_reduce_population_count(mskb)[0]
            return pos + 2 * _L, cnt2

        _, cnt = lax.while_loop(cond, body, (jnp.int32(0), jnp.int32(0)))

        first = plsc.load_gather(idxv, [zeros])
        for h in range(_K // _L):
            kv = iota + h * _L
            v = idxv[pl.ds(h * _L, _L)]
            v = jnp.where(kv < cnt, v, first)
            mv = jnp.full((_L,), ml, jnp.int32)
            gx = plsc.load_gather(frows, [v]) - cxv
            gy = plsc.load_gather(frows, [v + _N]) - cyv
            gz = plsc.load_gather(frows, [v + 2 * _N]) - czv
            plsc.store_scatter(obuf, [kv, mv], gx)
            plsc.store_scatter(obuf, [kv + _K, mv], gy)
            plsc.store_scatter(obuf, [kv + 2 * _K, mv], gz)
            tv = ((v >> 7) << 10) + (v & 127)
            idxv[pl.ds(64 + ml * _K + h * _L, _L)] = tv
        return carry

    lax.fori_loop(0, _MPT, per_center, 0)

    pltpu.sync_copy(idxv.at[pl.ds(64, _MPT * _K)],
                    idx_out.at[pl.ds(b * _MK + ms * _K, _MPT * _K)])
    for c3 in range(3):
        pltpu.sync_copy(obuf.at[pl.ds(c3 * _K, _K), :],
                        out4.at[b, c3, :, pl.ds(ms, _MPT)])

    plsc.subcore_barrier()

    svec = iota * _K
    c0 = cg * 16

    for t in range(2):
        band = cg * 2 + t
        pltpu.sync_copy(ff.at[pl.ds((b * 16 + band) * 8 * _N, 8 * _N)], frows)
        for p in range(2):
            def chunk(ci, carry):
                par = ci & 1

                @pl.when((ci & 3) == 0)
                def _():
                    pltpu.sync_copy(
                        idx_out.at[pl.ds(b * _MK + (ci >> 2) * 16384, 16384)],
                        idxv)

                @pl.when(ci >= 2)
                def _():
                    for _q in range(4):
                        pltpu.make_async_copy(
                            obuf.at[pl.ds(0, _K), :],
                            out4.at[b, 3, :, pl.ds(0, _MPT)],
                            sem_out).wait()

                base_c = (ci & 3) * 4096

                def unit(u, carry2):
                    k = u >> 3
                    mseg = u & 7
                    iv = plsc.load_gather(idxv, [svec + (base_c + mseg * 512 + k)])
                    for q in range(4):
                        row = par * 128 + q * _K + k
                        obuf[row, pl.ds(mseg * _L, _L)] = (
                            plsc.load_gather(frows, [iv + (p * 4 + q) * 128]))
                    return carry2

                lax.fori_loop(0, 256, unit, 0)

                for q in range(4):
                    ch = c0 + t * 8 + p * 4 + q
                    pltpu.async_copy(
                        obuf.at[pl.ds(par * 128 + q * _K, _K), :],
                        out4.at[b, 3 + ch, :, pl.ds(ci * _MPT, _MPT)],
                        sem_out)
                return carry

            lax.fori_loop(0, _M // _MPT, chunk, 0)
            for _q in range(8):
                pltpu.make_async_copy(
                    obuf.at[pl.ds(0, _K), :],
                    out4.at[b, 3, :, pl.ds(0, _MPT)],
                    sem_out).wait()


_fused = pl.kernel(
    _body,
    out_type=(
        jax.ShapeDtypeStruct((_B * _MK,), jnp.int32),
        jax.ShapeDtypeStruct((_B, _NCH, _K, _M), jnp.float32),
    ),
    mesh=_mesh,
    compiler_params=pltpu.CompilerParams(needs_layout_passes=False),
    scratch_types=[
        pltpu.VMEM((8 * _N,), jnp.float32),
        pltpu.VMEM((16384,), jnp.int32),
        pltpu.VMEM((256, _MPT), jnp.float32),
        pltpu.SemaphoreType.DMA,
    ],
)


def kernel(points_xyz, center_xyz, features):
    ptsT = jnp.transpose(points_xyz, (0, 2, 1)).reshape(-1)
    cenT = jnp.transpose(center_xyz, (0, 2, 1)).reshape(-1)
    ff = (features.reshape(_B, 16, 8, 64, 128)
          .transpose(0, 1, 3, 2, 4).reshape(-1))
    _, out4 = _fused(ptsT, cenT, ff)
    return jnp.swapaxes(out4, 2, 3)


# speedup vs baseline: 43.9109x; 1.8104x over previous
import jax
import jax.numpy as jnp
from jax import lax
from jax.experimental import pallas as pl
from jax.experimental.pallas import tpu as pltpu
from jax.experimental.pallas import tpu_sc as plsc

_B, _N, _M, _K, _C = 4, 8192, 1024, 32, 128
_R2 = 0.2 * 0.2
_NC, _NS, _L = 2, 16, 16
_TPB = (_NC * _NS) // _B
_MPT = _M // _TPB
_MK = _M * _K
_NCH = 3 + _C
_PLANE = _K * _M

_PX, _PY, _PZ = 0, _N, 2 * _N
_CX = 3 * _N

_mesh = plsc.VectorSubcoreMesh(core_axis_name="c", subcore_axis_name="s")


def _body(ptsT, cenT, ff, idx_out, out, frows, idxv, obuf, pbuf, pbuf2, sem_out):
    wid = lax.axis_index("c") * _NS + lax.axis_index("s")
    b = wid // _TPB
    cg = wid % _TPB
    ms = cg * _MPT

    iota = lax.iota(jnp.int32, _L)
    zeros = jnp.zeros((_L,), jnp.int32)

    pltpu.sync_copy(ptsT.at[pl.ds((b * 3 + 0) * _N, _N)], frows.at[pl.ds(_PX, _N)])
    pltpu.sync_copy(ptsT.at[pl.ds((b * 3 + 1) * _N, _N)], frows.at[pl.ds(_PY, _N)])
    pltpu.sync_copy(ptsT.at[pl.ds((b * 3 + 2) * _N, _N)], frows.at[pl.ds(_PZ, _N)])
    for c3 in range(3):
        pltpu.sync_copy(cenT.at[pl.ds((b * 3 + c3) * _M + ms, _MPT)],
                        frows.at[pl.ds(_CX + c3 * _MPT, _MPT)])

    physk = [((iota + h * _L) >> 3) * 1024 + ((iota + h * _L) & 7) * 128
             for h in range(_K // _L)]
    kmul = [(iota + h * _L) * _MPT for h in range(_K // _L)]

    def scan_pts(pos, cxv, cyv, czv):
        dx = frows[pl.ds(_PX + pos, _L)] - cxv
        dy = frows[pl.ds(_PY + pos, _L)] - cyv
        dz = frows[pl.ds(_PZ + pos, _L)] - czv
        return dx * dx + dy * dy + dz * dz < _R2

    def fixup(ml, cnt, base, cxv, cyv, czv):
        first = plsc.load_gather(pbuf, [jnp.full((_L,), base, jnp.int32)])
        for h in range(_K // _L):
            kv = iota + h * _L
            v = pbuf[pl.ds(base + h * _L, _L)]
            v = jnp.where(kv < cnt, v, first)
            gx = plsc.load_gather(frows, [v]) - cxv
            gy = plsc.load_gather(frows, [v + _N]) - cyv
            gz = plsc.load_gather(frows, [v + 2 * _N]) - czv
            addr = physk[h] + ml
            plsc.store_scatter(obuf, [addr], gx)
            plsc.store_scatter(obuf, [addr + 4096], gy)
            plsc.store_scatter(obuf, [addr + 8192], gz)
            tv = ((v >> 7) << 10) + (v & 127)
            plsc.store_scatter(pbuf2, [kmul[h] + ml], tv)

    def per_pair(pi, carry):
        ml1 = 2 * pi
        ml2 = ml1 + 1
        c1x = plsc.load_gather(frows, [jnp.full((_L,), _CX + ml1, jnp.int32)])
        c1y = plsc.load_gather(frows, [jnp.full((_L,), _CX + _MPT + ml1, jnp.int32)])
        c1z = plsc.load_gather(frows, [jnp.full((_L,), _CX + 2 * _MPT + ml1, jnp.int32)])
        c2x = plsc.load_gather(frows, [jnp.full((_L,), _CX + ml2, jnp.int32)])
        c2y = plsc.load_gather(frows, [jnp.full((_L,), _CX + _MPT + ml2, jnp.int32)])
        c2z = plsc.load_gather(frows, [jnp.full((_L,), _CX + 2 * _MPT + ml2, jnp.int32)])
        pbuf[pl.ds(0, _L)] = zeros
        pbuf[pl.ds(64, _L)] = zeros

        def cond(st):
            pos, c1, c2 = st
            return jnp.logical_and(
                jnp.logical_or(c1 < _K, c2 < _K), pos < _N)

        def body(st):
            pos, c1, c2 = st
            ia = iota + pos
            ib = iota + (pos + _L)
            ma1 = scan_pts(pos, c1x, c1y, c1z)
            ma2 = scan_pts(pos, c2x, c2y, c2z)
            plsc.store_compressed(pbuf.at[pl.ds(jnp.minimum(c1, 48), _L)],
                                  ia, mask=ma1)
            plsc.store_compressed(pbuf.at[pl.ds(64 + jnp.minimum(c2, 48), _L)],
                                  ia, mask=ma2)
            c1a = c1 + plsc.all_reduce_population_count(ma1)[0]
            c2a = c2 + plsc.all_reduce_population_count(ma2)[0]
            mb1 = scan_pts(pos + _L, c1x, c1y, c1z)
            mb2 = scan_pts(pos + _L, c2x, c2y, c2z)
            plsc.store_compressed(pbuf.at[pl.ds(jnp.minimum(c1a, 48), _L)],
                                  ib, mask=mb1)
            plsc.store_compressed(pbuf.at[pl.ds(64 + jnp.minimum(c2a, 48), _L)],
                                  ib, mask=mb2)
            c1b = c1a + plsc.all_reduce_population_count(mb1)[0]
            c2b = c2a + plsc.all_reduce_population_count(mb2)[0]
            return pos + 2 * _L, c1b, c2b

        _, c1, c2 = lax.while_loop(
            cond, body, (jnp.int32(0), jnp.int32(0), jnp.int32(0)))

        fixup(ml1, c1, 0, c1x, c1y, c1z)
        fixup(ml2, c2, 64, c2x, c2y, c2z)
        return carry

    lax.fori_loop(0, _MPT // 2, per_pair, 0)

    pltpu.sync_copy(pbuf2, idx_out.at[pl.ds((b * _TPB + cg) * 4096, 4096)])
    for c3 in range(3):
        for tk in range(4):
            dst = (b * _NCH + c3) * _PLANE + tk * 8192 + cg * 1024
            pltpu.sync_copy(obuf.at[pl.ds(c3 * 4096 + tk * 1024, 1024)],
                            out.at[pl.ds(dst, 1024)])

    plsc.subcore_barrier()

    c0 = cg * 16

    for t in range(2):
        band = cg * 2 + t
        pltpu.sync_copy(ff.at[pl.ds((b * 16 + band) * 8 * _N, 8 * _N)], frows)
        for p in range(2):
            fr = [frows.at[pl.ds((p * 4 + q) * 128,
                                 8 * _N - (p * 4 + q) * 128)]
                  for q in range(4)]

            def chunk(ci, carry):
                par = ci & 1

                @pl.when((ci & 3) == 0)
                def _():
                    pltpu.sync_copy(
                        idx_out.at[pl.ds(b * 8 * 4096 + (ci >> 2) * 16384,
                                         16384)], idxv)

                @pl.when(ci >= 2)
                def _():
                    for _q in range(16):
                        pltpu.make_async_copy(
                            obuf.at[pl.ds(0, 1024)],
                            out.at[pl.ds(0, 1024)],
                            sem_out).wait()

                ib0 = (ci & 3) * 4096
                ob0 = par * 16384

                def kbody(k, carry2):
                    kb = ib0 + k * _MPT
                    sb = ob0 + (k >> 3) * 1024 + (k & 7) * 128
                    iv = idxv[pl.ds(kb, _L)]
                    for mseg in range(8):
                        vals = [plsc.load_gather(fr[q], [iv]) for q in range(4)]
                        if mseg < 7:
                            iv = idxv[pl.ds(kb + (mseg + 1) * _L, _L)]
                        for q in range(4):
                            obuf[pl.ds(sb + q * 4096 + mseg * _L, _L)] = vals[q]
                    return carry2

                lax.fori_loop(0, _K, kbody, 0)

                for q in range(4):
                    ch = c0 + t * 8 + p * 4 + q
                    pb = (b * _NCH + 3 + ch) * _PLANE + ci * 1024
                    for tk in range(4):
                        pltpu.async_copy(
                            obuf.at[pl.ds(ob0 + q * 4096 + tk * 1024, 1024)],
                            out.at[pl.ds(pb + tk * 8192, 1024)],
                            sem_out)
                return carry

            lax.fori_loop(0, _M // _MPT, chunk, 0)
            for _q in range(32):
                pltpu.make_async_copy(
                    obuf.at[pl.ds(0, 1024)],
                    out.at[pl.ds(0, 1024)],
                    sem_out).wait()


_fused = pl.kernel(
    _body,
    out_type=(
        jax.ShapeDtypeStruct((_B * _TPB * 4096,), jnp.int32),
        jax.ShapeDtypeStruct((_B * _NCH * _PLANE,), jnp.float32),
    ),
    mesh=_mesh,
    compiler_params=pltpu.CompilerParams(needs_layout_passes=False),
    scratch_types=[
        pltpu.VMEM((8 * _N,), jnp.float32),
        pltpu.VMEM((16384,), jnp.int32),
        pltpu.VMEM((32768,), jnp.float32),
        pltpu.VMEM((128,), jnp.int32),
        pltpu.VMEM((4096,), jnp.int32),
        pltpu.SemaphoreType.DMA,
    ],
)


def kernel(points_xyz, center_xyz, features):
    ptsT = jnp.transpose(points_xyz, (0, 2, 1)).reshape(-1)
    cenT = jnp.transpose(center_xyz, (0, 2, 1)).reshape(-1)
    ff = (features.reshape(_B, 16, 8, 64, 128)
          .transpose(0, 1, 3, 2, 4).reshape(-1))
    _, out = _fused(ptsT, cenT, ff)
    o6 = out.reshape(_B, _NCH, 4, 8, 8, 128)
    return o6.transpose(0, 1, 3, 5, 2, 4).reshape(_B, _NCH, _M, _K)


# speedup vs baseline: 43.9442x; 1.0008x over previous
import jax
import jax.numpy as jnp
from jax import lax
from jax.experimental import pallas as pl
from jax.experimental.pallas import tpu as pltpu
from jax.experimental.pallas import tpu_sc as plsc

_B, _N, _M, _K, _C = 4, 8192, 1024, 32, 128
_R2 = 0.2 * 0.2
_NC, _NS, _L = 2, 16, 16
_TPB = (_NC * _NS) // _B
_MPT = _M // _TPB
_MK = _M * _K
_NCH = 3 + _C
_PLANE = _K * _M

_PX, _PY, _PZ = 0, _N, 2 * _N
_CX = 3 * _N

_mesh = plsc.VectorSubcoreMesh(core_axis_name="c", subcore_axis_name="s")


def _body(ptsT, cenT, ff, idx_out, out, frows, idxv, obuf, pbuf, pbuf2, sem_out):
    wid = lax.axis_index("c") * _NS + lax.axis_index("s")
    b = wid // _TPB
    cg = wid % _TPB
    ms = cg * _MPT

    iota = lax.iota(jnp.int32, _L)
    zeros = jnp.zeros((_L,), jnp.int32)
    full = jnp.ones((_L,), jnp.bool_)

    pltpu.sync_copy(ptsT.at[pl.ds((b * 3 + 0) * _N, _N)], frows.at[pl.ds(_PX, _N)])
    pltpu.sync_copy(ptsT.at[pl.ds((b * 3 + 1) * _N, _N)], frows.at[pl.ds(_PY, _N)])
    pltpu.sync_copy(ptsT.at[pl.ds((b * 3 + 2) * _N, _N)], frows.at[pl.ds(_PZ, _N)])
    for c3 in range(3):
        pltpu.sync_copy(cenT.at[pl.ds((b * 3 + c3) * _M + ms, _MPT)],
                        frows.at[pl.ds(_CX + c3 * _MPT, _MPT)])

    physk = [((iota + h * _L) >> 3) * 1024 + ((iota + h * _L) & 7) * 128
             for h in range(_K // _L)]
    kmul = [(iota + h * _L) * _MPT for h in range(_K // _L)]

    def scan_pts(pos, cxv, cyv, czv):
        dx = frows[pl.ds(_PX + pos, _L)] - cxv
        dy = frows[pl.ds(_PY + pos, _L)] - cyv
        dz = frows[pl.ds(_PZ + pos, _L)] - czv
        return dx * dx + dy * dy + dz * dz < _R2

    def fixup(ml, cnt, base, cxv, cyv, czv):
        first = plsc.load_gather(pbuf, [jnp.full((_L,), base, jnp.int32)])
        for h in range(_K // _L):
            kv = iota + h * _L
            v = pbuf[pl.ds(base + h * _L, _L)]
            v = jnp.where(kv < cnt, v, first)
            gx = plsc.load_gather(frows, [v]) - cxv
            gy = plsc.load_gather(frows, [v + _N]) - cyv
            gz = plsc.load_gather(frows, [v + 2 * _N]) - czv
            addr = physk[h] + ml
            plsc.store_scatter(obuf, [addr], gx)
            plsc.store_scatter(obuf, [addr + 4096], gy)
            plsc.store_scatter(obuf, [addr + 8192], gz)
            tv = ((v >> 7) << 10) + (v & 127)
            plsc.store_scatter(pbuf2, [kmul[h] + ml], tv)

    def per_pair(pi, carry):
        ml1 = 2 * pi
        ml2 = ml1 + 1
        c1x = plsc.load_gather(frows, [jnp.full((_L,), _CX + ml1, jnp.int32)])
        c1y = plsc.load_gather(frows, [jnp.full((_L,), _CX + _MPT + ml1, jnp.int32)])
        c1z = plsc.load_gather(frows, [jnp.full((_L,), _CX + 2 * _MPT + ml1, jnp.int32)])
        c2x = plsc.load_gather(frows, [jnp.full((_L,), _CX + ml2, jnp.int32)])
        c2y = plsc.load_gather(frows, [jnp.full((_L,), _CX + _MPT + ml2, jnp.int32)])
        c2z = plsc.load_gather(frows, [jnp.full((_L,), _CX + 2 * _MPT + ml2, jnp.int32)])
        pbuf[pl.ds(0, _L)] = zeros
        pbuf[pl.ds(64, _L)] = zeros

        def cond(st):
            pos, c1, c2 = st
            return jnp.logical_and(
                jnp.logical_or(c1 < _K, c2 < _K), pos < _N)

        def body(st):
            pos, c1, c2 = st
            ia = iota + pos
            ib = iota + (pos + _L)
            ma1 = scan_pts(pos, c1x, c1y, c1z)
            ma2 = scan_pts(pos, c2x, c2y, c2z)
            plsc.store_compressed(pbuf.at[pl.ds(jnp.minimum(c1, 48), _L)],
                                  ia, mask=ma1)
            plsc.store_compressed(pbuf.at[pl.ds(64 + jnp.minimum(c2, 48), _L)],
                                  ia, mask=ma2)
            c1a = c1 + plsc.all_reduce_population_count(ma1)[0]
            c2a = c2 + plsc.all_reduce_population_count(ma2)[0]
            mb1 = scan_pts(pos + _L, c1x, c1y, c1z)
            mb2 = scan_pts(pos + _L, c2x, c2y, c2z)
            plsc.store_compressed(pbuf.at[pl.ds(jnp.minimum(c1a, 48), _L)],
                                  ib, mask=mb1)
            plsc.store_compressed(pbuf.at[pl.ds(64 + jnp.minimum(c2a, 48), _L)],
                                  ib, mask=mb2)
            c1b = c1a + plsc.all_reduce_population_count(mb1)[0]
            c2b = c2a + plsc.all_reduce_population_count(mb2)[0]
            return pos + 2 * _L, c1b, c2b

        _, c1, c2 = lax.while_loop(
            cond, body, (jnp.int32(0), jnp.int32(0), jnp.int32(0)))

        fixup(ml1, c1, 0, c1x, c1y, c1z)
        fixup(ml2, c2, 64, c2x, c2y, c2z)
        return carry

    lax.fori_loop(0, _MPT // 2, per_pair, 0)

    pltpu.sync_copy(pbuf2, idx_out.at[pl.ds((b * _TPB + cg) * 4096, 4096)])
    for c3 in range(3):
        for tk in range(4):
            dst = (b * _NCH + c3) * _PLANE + tk * 8192 + cg * 1024
            pltpu.sync_copy(obuf.at[pl.ds(c3 * 4096 + tk * 1024, 1024)],
                            out.at[pl.ds(dst, 1024)])

    plsc.subcore_barrier()

    c0 = cg * 16

    for t in range(2):
        band = cg * 2 + t
        pltpu.sync_copy(ff.at[pl.ds((b * 16 + band) * 8 * _N, 8 * _N)], frows)
        for p in range(2):
            fr = [frows.at[pl.ds((p * 4 + q) * 128,
                                 8 * _N - (p * 4 + q) * 128)]
                  for q in range(4)]

            def chunk(ci, carry):
                par = ci & 1

                @pl.when((ci & 3) == 0)
                def _():
                    pltpu.sync_copy(
                        idx_out.at[pl.ds(b * 8 * 4096 + (ci >> 2) * 16384,
                                         16384)], idxv)

                @pl.when(ci >= 2)
                def _():
                    for _q in range(16):
                        pltpu.make_async_copy(
                            obuf.at[pl.ds(0, 1024)],
                            out.at[pl.ds(0, 1024)],
                            sem_out).wait()

                ib0 = (ci & 3) * 4096
                ob0 = par * 16384

                def kbody(k, carry2):
                    kb = ib0 + k * _MPT
                    sb = ob0 + (k >> 3) * 1024 + (k & 7) * 128
                    iv = plsc.load_expanded(idxv.at[pl.ds(kb, _L)], mask=full)
                    for mseg in range(8):
                        vals = [plsc.load_gather(fr[q], [iv]) for q in range(4)]
                        if mseg < 7:
                            iv = plsc.load_expanded(
                                idxv.at[pl.ds(kb + (mseg + 1) * _L, _L)],
                                mask=full)
                        for q in range(4):
                            plsc.store_compressed(
                                obuf.at[pl.ds(sb + q * 4096 + mseg * _L, _L)],
                                vals[q], mask=full)
                    return carry2

                lax.fori_loop(0, _K, kbody, 0)

                for q in range(4):
                    ch = c0 + t * 8 + p * 4 + q
                    pb = (b * _NCH + 3 + ch) * _PLANE + ci * 1024
                    for tk in range(4):
                        pltpu.async_copy(
                            obuf.at[pl.ds(ob0 + q * 4096 + tk * 1024, 1024)],
                            out.at[pl.ds(pb + tk * 8192, 1024)],
                            sem_out)
                return carry

            lax.fori_loop(0, _M // _MPT, chunk, 0)
            for _q in range(32):
                pltpu.make_async_copy(
                    obuf.at[pl.ds(0, 1024)],
                    out.at[pl.ds(0, 1024)],
                    sem_out).wait()


_fused = pl.kernel(
    _body,
    out_type=(
        jax.ShapeDtypeStruct((_B * _TPB * 4096,), jnp.int32),
        jax.ShapeDtypeStruct((_B * _NCH * _PLANE,), jnp.float32),
    ),
    mesh=_mesh,
    compiler_params=pltpu.CompilerParams(needs_layout_passes=False,
                                         disable_bounds_checks=True),
    scratch_types=[
        pltpu.VMEM((8 * _N,), jnp.float32),
        pltpu.VMEM((16384,), jnp.int32),
        pltpu.VMEM((32768,), jnp.float32),
        pltpu.VMEM((128,), jnp.int32),
        pltpu.VMEM((4096,), jnp.int32),
        pltpu.SemaphoreType.DMA,
    ],
)


def kernel(points_xyz, center_xyz, features):
    ptsT = jnp.transpose(points_xyz, (0, 2, 1)).reshape(-1)
    cenT = jnp.transpose(center_xyz, (0, 2, 1)).reshape(-1)
    ff = (features.reshape(_B, 16, 8, 64, 128)
          .transpose(0, 1, 3, 2, 4).reshape(-1))
    _, out = _fused(ptsT, cenT, ff)
    o6 = out.reshape(_B, _NCH, 4, 8, 8, 128)
    return o6.transpose(0, 1, 3, 5, 2, 4).reshape(_B, _NCH, _M, _K)


# speedup vs baseline: 49.7395x; 1.1319x over previous
import jax
import jax.numpy as jnp
from jax import lax
from jax.experimental import pallas as pl
from jax.experimental.pallas import tpu as pltpu
from jax.experimental.pallas import tpu_sc as plsc

_B, _N, _M, _K, _C = 4, 8192, 1024, 32, 128
_R2 = 0.2 * 0.2
_NC, _NS, _L = 2, 16, 16
_TPB = (_NC * _NS) // _B
_MPT = _M // _TPB
_MK = _M * _K
_NCH = 3 + _C
_PLANE = _K * _M

_PX, _PY, _PZ = 0, _N, 2 * _N
_CX = 3 * _N

_mesh = plsc.VectorSubcoreMesh(core_axis_name="c", subcore_axis_name="s")


def _body(ptsT, cenT, ff, idx_out, out, frows, idxv, obuf, pbuf, pbuf2, sem_out):
    wid = lax.axis_index("c") * _NS + lax.axis_index("s")
    b = wid // _TPB
    cg = wid % _TPB
    ms = cg * _MPT

    iota = lax.iota(jnp.int32, _L)
    zeros = jnp.zeros((_L,), jnp.int32)
    full = jnp.ones((_L,), jnp.bool_)

    pltpu.sync_copy(ptsT.at[pl.ds((b * 3 + 0) * _N, _N)], frows.at[pl.ds(_PX, _N)])
    pltpu.sync_copy(ptsT.at[pl.ds((b * 3 + 1) * _N, _N)], frows.at[pl.ds(_PY, _N)])
    pltpu.sync_copy(ptsT.at[pl.ds((b * 3 + 2) * _N, _N)], frows.at[pl.ds(_PZ, _N)])
    for c3 in range(3):
        pltpu.sync_copy(cenT.at[pl.ds((b * 3 + c3) * _M + ms, _MPT)],
                        frows.at[pl.ds(_CX + c3 * _MPT, _MPT)])

    physk = [((iota + h * _L) >> 3) * 1024 + ((iota + h * _L) & 7) * 128
             for h in range(_K // _L)]
    kmul = [(iota + h * _L) * _MPT for h in range(_K // _L)]

    def scan_pts(pos, cxv, cyv, czv):
        dx = frows[pl.ds(_PX + pos, _L)] - cxv
        dy = frows[pl.ds(_PY + pos, _L)] - cyv
        dz = frows[pl.ds(_PZ + pos, _L)] - czv
        return dx * dx + dy * dy + dz * dz < _R2

    def fixup(ml, cnt, base, cxv, cyv, czv):
        first = plsc.load_gather(pbuf, [jnp.full((_L,), base, jnp.int32)])
        for h in range(_K // _L):
            kv = iota + h * _L
            v = pbuf[pl.ds(base + h * _L, _L)]
            v = jnp.where(kv < cnt, v, first)
            gx = plsc.load_gather(frows, [v]) - cxv
            gy = plsc.load_gather(frows, [v + _N]) - cyv
            gz = plsc.load_gather(frows, [v + 2 * _N]) - czv
            addr = physk[h] + ml
            plsc.store_scatter(obuf, [addr], gx)
            plsc.store_scatter(obuf, [addr + 4096], gy)
            plsc.store_scatter(obuf, [addr + 8192], gz)
            tv = ((v >> 7) << 10) + (v & 127)
            plsc.store_scatter(pbuf2, [kmul[h] + ml], tv)

    def per_quad(pi, carry):
        mls = [4 * pi + j for j in range(4)]
        cs = []
        for ml in mls:
            cs.append((
                plsc.load_gather(frows, [jnp.full((_L,), _CX + ml, jnp.int32)]),
                plsc.load_gather(frows, [jnp.full((_L,), _CX + _MPT + ml,
                                                  jnp.int32)]),
                plsc.load_gather(frows, [jnp.full((_L,), _CX + 2 * _MPT + ml,
                                                  jnp.int32)]),
            ))
        for j in range(4):
            pbuf[pl.ds(64 * j, _L)] = zeros

        def cond(st):
            pos = st[0]
            unfinished = (st[1] < _K) | (st[2] < _K) | (st[3] < _K) | (st[4] < _K)
            return jnp.logical_and(unfinished, pos < _N)

        def body(st):
            pos = st[0]
            cnts = list(st[1:])
            for h in range(2):
                ih = iota + (pos + h * _L)
                ms = [scan_pts(pos + h * _L, *cs[j]) for j in range(4)]
                for j in range(4):
                    plsc.store_compressed(
                        pbuf.at[pl.ds(64 * j + jnp.minimum(cnts[j], 48), _L)],
                        ih, mask=ms[j])
                for j in range(4):
                    cnts[j] = cnts[j] + plsc.all_reduce_population_count(ms[j])[0]
            return (pos + 2 * _L, *cnts)

        st = lax.while_loop(
            cond, body, (jnp.int32(0),) + (jnp.int32(0),) * 4)

        for j in range(4):
            fixup(mls[j], st[1 + j], 64 * j, *cs[j])
        return carry

    lax.fori_loop(0, _MPT // 4, per_quad, 0)

    pltpu.sync_copy(pbuf2, idx_out.at[pl.ds((b * _TPB + cg) * 4096, 4096)])
    for c3 in range(3):
        for tk in range(4):
            dst = (b * _NCH + c3) * _PLANE + tk * 8192 + cg * 1024
            pltpu.sync_copy(obuf.at[pl.ds(c3 * 4096 + tk * 1024, 1024)],
                            out.at[pl.ds(dst, 1024)])

    plsc.subcore_barrier()

    c0 = cg * 16

    for t in range(2):
        band = cg * 2 + t
        pltpu.sync_copy(ff.at[pl.ds((b * 16 + band) * 8 * _N, 8 * _N)], frows)
        for p in range(2):
            fr = [frows.at[pl.ds((p * 4 + q) * 128,
                                 8 * _N - (p * 4 + q) * 128)]
                  for q in range(4)]

            def chunk(ci, carry):
                par = ci & 1

                @pl.when((ci & 3) == 0)
                def _():
                    pltpu.sync_copy(
                        idx_out.at[pl.ds(b * 8 * 4096 + (ci >> 2) * 16384,
                                         16384)], idxv)

                @pl.when(ci >= 2)
                def _():
                    for _q in range(16):
                        pltpu.make_async_copy(
                            obuf.at[pl.ds(0, 1024)],
                            out.at[pl.ds(0, 1024)],
                            sem_out).wait()

                ib0 = (ci & 3) * 4096
                ob0 = par * 16384

                def kbody(k, carry2):
                    kb = ib0 + k * _MPT
                    sb = ob0 + (k >> 3) * 1024 + (k & 7) * 128
                    iv = plsc.load_expanded(idxv.at[pl.ds(kb, _L)], mask=full)
                    for mseg in range(8):
                        vals = [plsc.load_gather(fr[q], [iv]) for q in range(4)]
                        if mseg < 7:
                            iv = plsc.load_expanded(
                                idxv.at[pl.ds(kb + (mseg + 1) * _L, _L)],
                                mask=full)
                        for q in range(4):
                            plsc.store_compressed(
                                obuf.at[pl.ds(sb + q * 4096 + mseg * _L, _L)],
                                vals[q], mask=full)
                    return carry2

                lax.fori_loop(0, _K, kbody, 0)

                for q in range(4):
                    ch = c0 + t * 8 + p * 4 + q
                    pb = (b * _NCH + 3 + ch) * _PLANE + ci * 1024
                    for tk in range(4):
                        pltpu.async_copy(
                            obuf.at[pl.ds(ob0 + q * 4096 + tk * 1024, 1024)],
                            out.at[pl.ds(pb + tk * 8192, 1024)],
                            sem_out)
                return carry

            lax.fori_loop(0, _M // _MPT, chunk, 0)
            for _q in range(32):
                pltpu.make_async_copy(
                    obuf.at[pl.ds(0, 1024)],
                    out.at[pl.ds(0, 1024)],
                    sem_out).wait()


_fused = pl.kernel(
    _body,
    out_type=(
        jax.ShapeDtypeStruct((_B * _TPB * 4096,), jnp.int32),
        jax.ShapeDtypeStruct((_B * _NCH * _PLANE,), jnp.float32),
    ),
    mesh=_mesh,
    compiler_params=pltpu.CompilerParams(needs_layout_passes=False,
                                         disable_bounds_checks=True),
    scratch_types=[
        pltpu.VMEM((8 * _N,), jnp.float32),
        pltpu.VMEM((16384,), jnp.int32),
        pltpu.VMEM((32768,), jnp.float32),
        pltpu.VMEM((256,), jnp.int32),
        pltpu.VMEM((4096,), jnp.int32),
        pltpu.SemaphoreType.DMA,
    ],
)


def kernel(points_xyz, center_xyz, features):
    ptsT = jnp.transpose(points_xyz, (0, 2, 1)).reshape(-1)
    cenT = jnp.transpose(center_xyz, (0, 2, 1)).reshape(-1)
    ff = (features.reshape(_B, 16, 8, 64, 128)
          .transpose(0, 1, 3, 2, 4).reshape(-1))
    _, out = _fused(ptsT, cenT, ff)
    o6 = out.reshape(_B, _NCH, 4, 8, 8, 128)
    return o6.transpose(0, 1, 3, 5, 2, 4).reshape(_B, _NCH, _M, _K)
